# baseline (speedup 1.0000x reference)
import functools

import jax
import jax.numpy as jnp
import numpy as np
from jax import lax
from jax.experimental import pallas as pl
from jax.experimental.pallas import tpu as pltpu
from jax.experimental.pallas import tpu_sc as plsc

N_NODES = 10000
N_EDGES = 320000
D = 128

NC = 2
NS = 16
NW = NC * NS
EPW = N_EDGES // NW
B = 80
NB = EPW // B
NBUF = 4
N_PAD = 10240
ROWS_PER_TILE = N_PAD // NS
DRAIN = 80
NDRAIN = ROWS_PER_TILE // DRAIN

_PERM = np.zeros((D,), np.int32)
for _k in range(D // 32):
    for _i in range(16):
        _PERM[32 * _k + _i] = 32 * _k + 2 * _i
        _PERM[32 * _k + 16 + _i] = 32 * _k + 2 * _i + 1


def _sc_segment_sum(x16, col, row, val):
    mesh = plsc.VectorSubcoreMesh(core_axis_name="c", subcore_axis_name="s")

    @functools.partial(
        pl.kernel,
        out_type=jax.ShapeDtypeStruct((NC, N_PAD, D), jnp.float32),
        mesh=mesh,
        compiler_params=pltpu.CompilerParams(use_tc_tiling_on_sc=False),
        scratch_types=[
            [pltpu.VMEM((B,), jnp.int32) for _ in range(NBUF)],
            [pltpu.VMEM((B,), jnp.int32) for _ in range(NBUF)],
            [pltpu.VMEM((B,), jnp.float32) for _ in range(NBUF)],
            [pltpu.VMEM((B, D // 2), jnp.int32) for _ in range(2)],
            [pltpu.VMEM((B, D), jnp.float32) for _ in range(2)],
            pltpu.VMEM_SHARED((N_PAD, D), jnp.float32),
            [pltpu.SemaphoreType.DMA for _ in range(NBUF)],
            [pltpu.SemaphoreType.DMA for _ in range(2)],
            [pltpu.SemaphoreType.DMA for _ in range(2)],
        ],
    )
    def seg_sum(x_hbm, col_hbm, row_hbm, val_hbm, out_hbm,
                cbufs, rbufs, vbufs, hbufs2, fbufs2, acc,
                isems, gsems2, ssems2):
        c = lax.axis_index("c")
        s = lax.axis_index("s")
        wid = s * NC + c
        base = wid * EPW

        hbufs = [hbufs2[0], hbufs2[1]] * 2
        fbufs = [fbufs2[0], fbufs2[1]] * 2
        gsems = [gsems2[0], gsems2[1]] * 2
        ssems = [ssems2[0], ssems2[1]] * 2

        zero16 = jnp.zeros((16,), jnp.float32)

        def zrow(r, _):
            for j in range(D // 16):
                fbufs[0][r, pl.ds(j * 16, 16)] = zero16
            return 0

        lax.fori_loop(0, DRAIN, zrow, 0)
        for k in range(NDRAIN):
            r0 = s * ROWS_PER_TILE + k * DRAIN
            pltpu.sync_copy(fbufs[0], acc.at[pl.ds(r0, DRAIN)])
        plsc.subcore_barrier()

        def idx_start(b, u):
            off = base + b * B
            pltpu.async_copy(col_hbm.at[pl.ds(off, B)], cbufs[u], isems[u])
            pltpu.async_copy(row_hbm.at[pl.ds(off, B)], rbufs[u], isems[u])
            pltpu.async_copy(val_hbm.at[pl.ds(off, B)], vbufs[u], isems[u])

        def idx_wait(b, u):
            off = base + b * B
            pltpu.make_async_copy(
                col_hbm.at[pl.ds(off, B)], cbufs[u], isems[u]).wait()
            pltpu.make_async_copy(
                row_hbm.at[pl.ds(off, B)], rbufs[u], isems[u]).wait()
            pltpu.make_async_copy(
                val_hbm.at[pl.ds(off, B)], vbufs[u], isems[u]).wait()

        def gather_start(b, u):
            pltpu.async_copy(x_hbm.at[cbufs[u]], hbufs[u], gsems[u])

        def gather_wait(b, u):
            pltpu.make_async_copy(
                x_hbm.at[cbufs[u]], hbufs[u], gsems[u]).wait()

        def scatter_start(b, u):
            pltpu.async_copy(fbufs[u], acc.at[rbufs[u]], ssems[u], add=True)

        def scatter_wait(b, u):
            pltpu.make_async_copy(
                fbufs[u], acc.at[rbufs[u]], ssems[u]).wait()

        dnums = lax.GatherDimensionNumbers(
            offset_dims=(), collapsed_slice_dims=(0,), start_index_map=(0,))

        def scale(b, u):
            hb = hbufs[u]
            fb = fbufs[u]
            vb = vbufs[u]

            def grp(g, _):
                e0 = g * 16
                vblock = vb[pl.ds(e0, 16)]

                def edge(e16, _):
                    v = lax.gather(
                        vblock,
                        jnp.full((16, 1), e16, jnp.int32),
                        dnums, (1,),
                        mode=lax.GatherScatterMode.PROMISE_IN_BOUNDS)
                    e = e0 + e16
                    for k in range(D // 32):
                        w = hb[e, pl.ds(16 * k, 16)]
                        lo = lax.bitcast_convert_type(w << 16, jnp.float32)
                        hi = lax.bitcast_convert_type(w & jnp.int32(-65536), jnp.float32)
                        fb[e, pl.ds(32 * k, 16)] = lo * v
                        fb[e, pl.ds(32 * k + 16, 16)] = hi * v
                    return 0

                lax.fori_loop(0, 16, edge, 0)
                return 0

            lax.fori_loop(0, B // 16, grp, 0)

        def stage(b, u, sw=True, di=True, dg=True):
            un, up = (u + 1) % NBUF, (u + 2) % NBUF
            if sw:
                scatter_wait(b - 2, up)
            if di:
                idx_start(b + 2, up)
            if dg:
                idx_wait(b + 1, un)
                gather_start(b + 1, un)
            gather_wait(b, u)
            scale(b, u)
            scatter_start(b, u)

        idx_start(0, 0)
        idx_start(1, 1)
        idx_wait(0, 0)
        gather_start(0, 0)

        stage(0, 0, sw=False)
        stage(1, 1, sw=False)
        stage(2, 2)
        stage(3, 3)

        def quad(i, _):
            for u in range(NBUF):
                stage(NBUF * i + u, u)
            return 0

        lax.fori_loop(1, (NB - 5) // NBUF, quad, 0)

        stage(NB - 5, 0)
        stage(NB - 4, 1)
        stage(NB - 3, 2)
        stage(NB - 2, 3, di=False)
        stage(NB - 1, 0, di=False, dg=False)

        scatter_wait(NB - 2, (NB - 2) % NBUF)
        scatter_wait(NB - 1, (NB - 1) % NBUF)
        plsc.subcore_barrier()

        for k in range(NDRAIN):
            r0 = s * ROWS_PER_TILE + k * DRAIN
            pltpu.sync_copy(acc.at[pl.ds(r0, DRAIN)],
                            out_hbm.at[c, pl.ds(r0, DRAIN)])

    return seg_sum(x16, col, row, val)


BLK = 1000


def _tc_body(p_ref, w_ref, o_ref):
    a = p_ref[0] + p_ref[1]
    o_ref[...] = jnp.dot(a, w_ref[...], preferred_element_type=jnp.float32)


def _tc_add_matmul(partials, Wp):
    return pl.pallas_call(
        _tc_body,
        grid=(N_NODES // BLK,),
        in_specs=[
            pl.BlockSpec((NC, BLK, D), lambda i: (0, i, 0)),
            pl.BlockSpec((D, D), lambda i: (0, 0)),
        ],
        out_specs=pl.BlockSpec((BLK, D), lambda i: (i, 0)),
        out_shape=jax.ShapeDtypeStruct((N_NODES, D), jnp.float32),
    )(partials, Wp)


@jax.jit
def kernel(x, edge_index, edge_vals, W):
    row = edge_index[0].astype(jnp.int32)
    col = edge_index[1].astype(jnp.int32)
    x16 = x.astype(jnp.bfloat16)
    xpk = lax.bitcast_convert_type(
        x16.reshape(N_NODES, D // 2, 2), jnp.int32)
    Wp = W[jnp.asarray(_PERM), :]
    partials = _sc_segment_sum(xpk, col, row, edge_vals)
    return _tc_add_matmul(partials, Wp)


# speedup vs baseline: 2.0038x; 2.0038x over previous
import functools

import jax
import jax.numpy as jnp
from jax import lax
from jax.experimental import pallas as pl
from jax.experimental.pallas import tpu as pltpu
from jax.experimental.pallas import tpu_sc as plsc

N_NODES = 10000
N_EDGES = 320000
D = 128

NC = 2
NS = 16
NW = NC * NS
EPW = N_EDGES // NW
B = 80
NB = EPW // B
NBUF = 4
N_PAD = 10240
ROWS_PER_TILE = N_PAD // NS
DRAIN = 80
NDRAIN = ROWS_PER_TILE // DRAIN


def _sc_segment_sum(x, col, row, val):
    mesh = plsc.VectorSubcoreMesh(core_axis_name="c", subcore_axis_name="s")

    @functools.partial(
        pl.kernel,
        out_type=jax.ShapeDtypeStruct((NC, N_PAD, D), jnp.float32),
        mesh=mesh,
        scratch_types=[
            [pltpu.VMEM((B,), jnp.int32) for _ in range(NBUF)],
            [pltpu.VMEM((B,), jnp.int32) for _ in range(NBUF)],
            [pltpu.VMEM((B,), jnp.float32) for _ in range(NBUF)],
            [pltpu.VMEM((B, D), jnp.float32) for _ in range(NBUF)],
            pltpu.VMEM_SHARED((N_PAD, D), jnp.float32),
            [pltpu.SemaphoreType.DMA for _ in range(NBUF)],
            [pltpu.SemaphoreType.DMA for _ in range(NBUF)],
            [pltpu.SemaphoreType.DMA for _ in range(NBUF)],
        ],
    )
    def seg_sum(x_hbm, col_hbm, row_hbm, val_hbm, out_hbm,
                cbufs, rbufs, vbufs, rows, acc, isems, gsems, ssems):
        c = lax.axis_index("c")
        s = lax.axis_index("s")
        wid = s * NC + c
        base = wid * EPW

        zero16 = jnp.zeros((16,), jnp.float32)

        def zrow(r, _):
            for j in range(D // 16):
                rows[0][r, pl.ds(j * 16, 16)] = zero16
            return 0

        lax.fori_loop(0, DRAIN, zrow, 0)
        for k in range(NDRAIN):
            r0 = s * ROWS_PER_TILE + k * DRAIN
            pltpu.sync_copy(rows[0], acc.at[pl.ds(r0, DRAIN)])
        plsc.subcore_barrier()

        def idx_start(b, u):
            off = base + b * B
            pltpu.async_copy(col_hbm.at[pl.ds(off, B)], cbufs[u], isems[u])
            pltpu.async_copy(row_hbm.at[pl.ds(off, B)], rbufs[u], isems[u])
            pltpu.async_copy(val_hbm.at[pl.ds(off, B)], vbufs[u], isems[u])

        def idx_wait(b, u):
            off = base + b * B
            pltpu.make_async_copy(
                col_hbm.at[pl.ds(off, B)], cbufs[u], isems[u]).wait()
            pltpu.make_async_copy(
                row_hbm.at[pl.ds(off, B)], rbufs[u], isems[u]).wait()
            pltpu.make_async_copy(
                val_hbm.at[pl.ds(off, B)], vbufs[u], isems[u]).wait()

        def gather_start(b, u):
            pltpu.async_copy(x_hbm.at[cbufs[u]], rows[u], gsems[u])

        def gather_wait(b, u):
            pltpu.make_async_copy(
                x_hbm.at[cbufs[u]], rows[u], gsems[u]).wait()

        def scatter_start(b, u):
            pltpu.async_copy(rows[u], acc.at[rbufs[u]], ssems[u], add=True)

        def scatter_wait(b, u):
            pltpu.make_async_copy(
                rows[u], acc.at[rbufs[u]], ssems[u]).wait()

        dnums = lax.GatherDimensionNumbers(
            offset_dims=(), collapsed_slice_dims=(0,), start_index_map=(0,))

        def scale(b, u):
            rr = rows[u]
            vb = vbufs[u]

            def grp(g, _):
                e0 = g * 16
                vblock = vb[pl.ds(e0, 16)]

                def edge(e16, _):
                    v = lax.gather(
                        vblock,
                        jnp.full((16, 1), e16, jnp.int32),
                        dnums, (1,),
                        mode=lax.GatherScatterMode.PROMISE_IN_BOUNDS)
                    e = e0 + e16
                    for j in range(D // 16):
                        rr[e, pl.ds(j * 16, 16)] = (
                            rr[e, pl.ds(j * 16, 16)] * v)
                    return 0

                lax.fori_loop(0, 16, edge, 0)
                return 0

            lax.fori_loop(0, B // 16, grp, 0)

        def stage(b, u, sw=True, di=True, dg=True):
            un, up = (u + 1) % NBUF, (u + 2) % NBUF
            if sw:
                scatter_wait(b - 2, (u + 2) % NBUF)
            if di:
                idx_start(b + 2, up)
            if dg:
                idx_wait(b + 1, un)
                gather_start(b + 1, un)
            gather_wait(b, u)
            scale(b, u)
            scatter_start(b, u)

        idx_start(0, 0)
        idx_start(1, 1)
        idx_wait(0, 0)
        gather_start(0, 0)

        stage(0, 0, sw=False)
        stage(1, 1, sw=False)
        stage(2, 2)
        stage(3, 3)

        def quad(i, _):
            for u in range(NBUF):
                stage(NBUF * i + u, u)
            return 0

        lax.fori_loop(1, (NB - 5) // NBUF, quad, 0)

        stage(NB - 5, 0)
        stage(NB - 4, 1)
        stage(NB - 3, 2)
        stage(NB - 2, 3, di=False)
        stage(NB - 1, 0, di=False, dg=False)

        scatter_wait(NB - 2, (NB - 2) % NBUF)
        scatter_wait(NB - 1, (NB - 1) % NBUF)
        plsc.subcore_barrier()

        r0 = s * ROWS_PER_TILE
        pltpu.sync_copy(acc.at[pl.ds(r0, ROWS_PER_TILE)],
                        out_hbm.at[c, pl.ds(r0, ROWS_PER_TILE)])

    return seg_sum(x, col, row, val)


BLK = 1000


def _tc_body(p_ref, w_ref, o_ref):
    a = p_ref[0] + p_ref[1]
    o_ref[...] = jnp.dot(a, w_ref[...], preferred_element_type=jnp.float32)


def _tc_add_matmul(partials, W):
    return pl.pallas_call(
        _tc_body,
        grid=(N_NODES // BLK,),
        in_specs=[
            pl.BlockSpec((NC, BLK, D), lambda i: (0, i, 0)),
            pl.BlockSpec((D, D), lambda i: (0, 0)),
        ],
        out_specs=pl.BlockSpec((BLK, D), lambda i: (i, 0)),
        out_shape=jax.ShapeDtypeStruct((N_NODES, D), jnp.float32),
    )(partials, W)


@jax.jit
def kernel(x, edge_index, edge_vals, W):
    row = edge_index[0].astype(jnp.int32)
    col = edge_index[1].astype(jnp.int32)
    partials = _sc_segment_sum(x, col, row, edge_vals)
    return _tc_add_matmul(partials, W)


# speedup vs baseline: 2.3366x; 1.1661x over previous
import functools

import jax
import jax.numpy as jnp
from jax import lax
from jax.experimental import pallas as pl
from jax.experimental.pallas import tpu as pltpu
from jax.experimental.pallas import tpu_sc as plsc

N_NODES = 10000
N_EDGES = 320000
D = 128

NC = 2
NS = 16
NW = NC * NS
EPW = N_EDGES // NW
B = 80
NB = EPW // B
NBUF = 4
N_PAD = 10240
ROWS_PER_TILE = N_PAD // NS
DRAIN = 80
NDRAIN = ROWS_PER_TILE // DRAIN


def _sc_segment_sum(x, col, row, val):
    mesh = plsc.VectorSubcoreMesh(core_axis_name="c", subcore_axis_name="s")

    @functools.partial(
        pl.kernel,
        out_type=jax.ShapeDtypeStruct((NC, N_PAD, D), jnp.float32),
        mesh=mesh,
        scratch_types=[
            [pltpu.VMEM((B,), jnp.int32) for _ in range(NBUF)],
            [pltpu.VMEM((B,), jnp.int32) for _ in range(NBUF)],
            [pltpu.VMEM((B,), jnp.float32) for _ in range(NBUF)],
            [pltpu.VMEM((B, D), jnp.float32) for _ in range(NBUF)],
            pltpu.VMEM_SHARED((N_PAD, D), jnp.float32),
            [pltpu.SemaphoreType.DMA for _ in range(NBUF)],
            [pltpu.SemaphoreType.DMA for _ in range(NBUF)],
            [pltpu.SemaphoreType.DMA for _ in range(NBUF)],
        ],
    )
    def seg_sum(x_hbm, col_hbm, row_hbm, val_hbm, out_hbm,
                cbufs, rbufs, vbufs, rows, acc, isems, gsems, ssems):
        c = lax.axis_index("c")
        s = lax.axis_index("s")
        wid = s * NC + c
        base = wid * EPW

        def idx_start(b, u):
            off = base + b * B
            pltpu.async_copy(col_hbm.at[pl.ds(off, B)], cbufs[u], isems[u])
            pltpu.async_copy(row_hbm.at[pl.ds(off, B)], rbufs[u], isems[u])
            pltpu.async_copy(val_hbm.at[pl.ds(off, B)], vbufs[u], isems[u])

        def idx_wait(b, u):
            off = base + b * B
            pltpu.make_async_copy(
                col_hbm.at[pl.ds(off, B)], cbufs[u], isems[u]).wait()
            pltpu.make_async_copy(
                row_hbm.at[pl.ds(off, B)], rbufs[u], isems[u]).wait()
            pltpu.make_async_copy(
                val_hbm.at[pl.ds(off, B)], vbufs[u], isems[u]).wait()

        def gather_start(b, u):
            pltpu.async_copy(x_hbm.at[cbufs[u]], rows[u], gsems[u])

        def gather_wait(b, u):
            pltpu.make_async_copy(
                x_hbm.at[cbufs[u]], rows[u], gsems[u]).wait()

        def scatter_start(b, u):
            pltpu.async_copy(rows[u], acc.at[rbufs[u]], ssems[u], add=True)

        def scatter_wait(b, u):
            pltpu.make_async_copy(
                rows[u], acc.at[rbufs[u]], ssems[u]).wait()

        dnums = lax.GatherDimensionNumbers(
            offset_dims=(), collapsed_slice_dims=(0,), start_index_map=(0,))

        def scale(b, u):
            rr = rows[u]
            vb = vbufs[u]

            def grp(g, _):
                e0 = g * 16
                vblock = vb[pl.ds(e0, 16)]

                @functools.partial(plsc.parallel_loop, 0, 16, unroll=2)
                def edge(e16):
                    v = lax.gather(
                        vblock,
                        jnp.full((16, 1), e16, jnp.int32),
                        dnums, (1,),
                        mode=lax.GatherScatterMode.PROMISE_IN_BOUNDS)
                    e = e0 + e16
                    for j in range(D // 16):
                        rr[e, pl.ds(j * 16, 16)] = (
                            rr[e, pl.ds(j * 16, 16)] * v)

                return 0

            lax.fori_loop(0, B // 16, grp, 0)

        def stage(b, u, sw=True, di=True, dg=True):
            un, up = (u + 1) % NBUF, (u + 2) % NBUF
            if sw:
                scatter_wait(b - 2, (u + 2) % NBUF)
            if di:
                idx_start(b + 2, up)
            if dg:
                idx_wait(b + 1, un)
                gather_start(b + 1, un)
            gather_wait(b, u)
            scale(b, u)
            scatter_start(b, u)

        idx_start(0, 0)
        idx_start(1, 1)
        idx_wait(0, 0)
        gather_start(0, 0)

        zero16 = jnp.zeros((16,), jnp.float32)

        def zrow(r, _):
            for j in range(D // 16):
                rows[3][r, pl.ds(j * 16, 16)] = zero16
            return 0

        lax.fori_loop(0, DRAIN, zrow, 0)
        for k in range(NDRAIN):
            r0 = s * ROWS_PER_TILE + k * DRAIN
            pltpu.sync_copy(rows[3], acc.at[pl.ds(r0, DRAIN)])
        plsc.subcore_barrier()

        stage(0, 0, sw=False)
        stage(1, 1, sw=False)
        stage(2, 2)
        stage(3, 3)

        def quad(i, _):
            for u in range(NBUF):
                stage(NBUF * i + u, u)
            return 0

        lax.fori_loop(1, (NB - 5) // NBUF, quad, 0)

        stage(NB - 5, 0)
        stage(NB - 4, 1)
        stage(NB - 3, 2)
        stage(NB - 2, 3, di=False)
        stage(NB - 1, 0, di=False, dg=False)

        scatter_wait(NB - 2, (NB - 2) % NBUF)
        scatter_wait(NB - 1, (NB - 1) % NBUF)
        plsc.subcore_barrier()

        r0 = s * ROWS_PER_TILE
        pltpu.sync_copy(acc.at[pl.ds(r0, ROWS_PER_TILE)],
                        out_hbm.at[c, pl.ds(r0, ROWS_PER_TILE)])

    return seg_sum(x, col, row, val)


BLK = 1000


def _tc_body(p_ref, w_ref, o_ref):
    a = p_ref[0] + p_ref[1]
    o_ref[...] = jnp.dot(a, w_ref[...], preferred_element_type=jnp.float32)


def _tc_add_matmul(partials, W):
    return pl.pallas_call(
        _tc_body,
        grid=(N_NODES // BLK,),
        in_specs=[
            pl.BlockSpec((NC, BLK, D), lambda i: (0, i, 0)),
            pl.BlockSpec((D, D), lambda i: (0, 0)),
        ],
        out_specs=pl.BlockSpec((BLK, D), lambda i: (i, 0)),
        out_shape=jax.ShapeDtypeStruct((N_NODES, D), jnp.float32),
    )(partials, W)


@jax.jit
def kernel(x, edge_index, edge_vals, W):
    row = edge_index[0].astype(jnp.int32)
    col = edge_index[1].astype(jnp.int32)
    partials = _sc_segment_sum(x, col, row, edge_vals)
    return _tc_add_matmul(partials, W)
